# baseline (speedup 1.0000x reference)
import functools

import jax
import jax.numpy as jnp
from jax.experimental import pallas as pl
from jax.experimental.pallas import tpu as pltpu
from jax.experimental.pallas import tpu_sc as plsc

_K = 64
_N = 256
_B = 8
_TAU = 1.0
_H = _N // 2

_NSC = 1
_NSUB = 32 // _NSC
_ROWS = _N // _NSUB


def _rne_bf16(d):
    di = jax.lax.bitcast_convert_type(d, jnp.int32)
    lsb = jax.lax.shift_right_logical(di, 16) & 1
    di = (di + 0x7FFF + lsb) & jnp.int32(-65536)
    return jax.lax.bitcast_convert_type(di, jnp.float32)


@functools.partial(
    pl.kernel,
    out_type=[
        jax.ShapeDtypeStruct((_NSC * _N, _N), jnp.float32),
        jax.ShapeDtypeStruct((32, 16), jnp.float32),
    ],
    mesh=plsc.VectorSubcoreMesh(core_axis_name="c", subcore_axis_name="s"),
    scratch_types=[
        pltpu.VMEM((_K, _N), jnp.float32),
        pltpu.VMEM((_K, 16), jnp.float32),
        pltpu.VMEM((_ROWS, _K, 16), jnp.float32),
        pltpu.VMEM((_ROWS, _N), jnp.float32),
        pltpu.VMEM((16,), jnp.float32),
    ],
)
def _sc_glcn(hft_ref, wb_ref, xb_ref, a_ref, p_ref, hft_v, wb_v, xb_v, av_v, s_v):
    cid = jax.lax.axis_index("c")
    sid = jax.lax.axis_index("s")
    wid = sid * 2 + cid
    b = wid // _NSUB
    i0 = (wid % _NSUB) * _ROWS
    pltpu.sync_copy(hft_ref.at[b], hft_v)
    pltpu.sync_copy(wb_ref, wb_v)
    pltpu.sync_copy(xb_ref.at[b, pl.ds(i0, _ROWS)], xb_v)
    lane = jax.lax.iota(jnp.int32, 16)
    s_total = jnp.zeros((16,), jnp.float32)
    for il in range(_ROWS):
        iglob = i0 + il

        def kbody(k, accs, _il=il):
            xv = xb_v[_il, k, :]
            ws = wb_v[k, :]
            out = []
            for jc in range(16):
                row = hft_v[k, pl.ds(jc * 16, 16)]
                d = _rne_bf16(jnp.abs(xv - row))
                out.append(accs[jc] + ws * d)
            return tuple(out)

        accs = jax.lax.fori_loop(
            0, _K, kbody,
            tuple(jnp.zeros((16,), jnp.float32) for _ in range(16)))
        for jc in range(16):
            acc = accs[jc]
            isdiag = (lane + jc * 16) == iglob
            hard = acc > 0.0
            aval = jnp.where(isdiag | hard, 1.0, 0.0).astype(jnp.float32)
            av_v[il, pl.ds(jc * 16, 16)] = aval
            u = jnp.exp(-jnp.abs(acc))
            t = u / (2.0 + u)
            p2 = t * t
            ser = 1.0 + p2 * (1.0 / 3.0 + p2 * (1.0 / 5.0 + p2 * (
                1.0 / 7.0 + p2 * (1.0 / 9.0 + p2 * (1.0 / 11.0)))))
            plog = -2.0 * t * ser
            s_total = s_total + jnp.where(isdiag, 0.0, plog)
    s_v[...] = s_total
    pltpu.sync_copy(s_v, p_ref.at[wid])
    pltpu.sync_copy(av_v, a_ref.at[pl.ds(b * _N + i0, _ROWS)])


def _tc_body(x_ref, xt_ref, w_ref, a_ref, p_ref):
    x = x_ref[0]
    xt = xt_ref[0]
    a00 = jnp.zeros((_H, _H), jnp.float32)
    a01 = jnp.zeros((_H, _H), jnp.float32)
    a11 = jnp.zeros((_H, _H), jnp.float32)
    for k in range(_K):
        c0 = x[0:_H, k:k + 1]
        c1 = x[_H:_N, k:k + 1]
        r0 = xt[k:k + 1, 0:_H]
        r1 = xt[k:k + 1, _H:_N]
        wk = w_ref[k, 0]
        d00 = jnp.abs(c0 - r0).astype(jnp.bfloat16).astype(jnp.float32)
        d01 = jnp.abs(c0 - r1).astype(jnp.bfloat16).astype(jnp.float32)
        d11 = jnp.abs(c1 - r1).astype(jnp.bfloat16).astype(jnp.float32)
        a00 = a00 + wk * d00
        a01 = a01 + wk * d01
        a11 = a11 + wk * d11
    ii = jax.lax.broadcasted_iota(jnp.int32, (_H, _H), 0)
    jj = jax.lax.broadcasted_iota(jnp.int32, (_H, _H), 1)
    diag = ii == jj

    def _finish(acc, on_diag):
        y = jax.nn.sigmoid(acc / _TAU)
        hard = y > 0.5
        plog = jnp.log(jnp.where(hard, y, 1.0 - y) + 1e-8)
        if on_diag:
            a_blk = jnp.where(diag | hard, 1.0, 0.0).astype(jnp.float32)
            s = jnp.sum(jnp.where(diag, 0.0, plog))
        else:
            a_blk = jnp.where(hard, 1.0, 0.0).astype(jnp.float32)
            s = jnp.sum(plog)
        return a_blk, s

    A00, s00 = _finish(a00, True)
    A01, s01 = _finish(a01, False)
    A11, s11 = _finish(a11, True)
    a_ref[0, 0:_H, 0:_H] = A00
    a_ref[0, 0:_H, _H:_N] = A01
    a_ref[0, _H:_N, 0:_H] = jnp.transpose(A01)
    a_ref[0, _H:_N, _H:_N] = A11
    p_ref[...] = (s00 + s11 + 2.0 * s01).reshape(1, 1, 1)


def kernel(h, a_link, rollout):
    hf = jax.lax.stop_gradient(h[:, :, :_K])
    hft = jnp.transpose(hf, (0, 2, 1))
    w_r = _rne_bf16(a_link)
    b_tc = _B - _NSC

    wb = jnp.broadcast_to(w_r, (_K, 16))
    xb = jnp.broadcast_to(hf[:_NSC, :, :, None], (_NSC, _N, _K, 16))
    a_sc, p_part = _sc_glcn(hft[:_NSC], wb, xb)
    probs_sc = jnp.sum(p_part.reshape(_NSC, -1), axis=-1)

    a_tc, probs_tc = pl.pallas_call(
        _tc_body,
        grid=(b_tc,),
        in_specs=[
            pl.BlockSpec((1, _N, _K), lambda i: (i, 0, 0)),
            pl.BlockSpec((1, _K, _N), lambda i: (i, 0, 0)),
            pl.BlockSpec(memory_space=pltpu.SMEM),
        ],
        out_specs=[
            pl.BlockSpec((1, _N, _N), lambda i: (i, 0, 0)),
            pl.BlockSpec((1, 1, 1), lambda i: (i, 0, 0)),
        ],
        out_shape=[
            jax.ShapeDtypeStruct((b_tc, _N, _N), jnp.float32),
            jax.ShapeDtypeStruct((b_tc, 1, 1), jnp.float32),
        ],
    )(hf[_NSC:], hft[_NSC:], w_r)

    a_out = jnp.concatenate([a_sc.reshape(_NSC, _N, _N), a_tc], axis=0)
    probs = jnp.concatenate([probs_sc, probs_tc[:, 0, 0]], axis=0)
    return (a_out, probs)


# speedup vs baseline: 2.1045x; 2.1045x over previous
import jax
import jax.numpy as jnp
from jax.experimental import pallas as pl
from jax.experimental.pallas import tpu as pltpu

_K = 64
_N = 256
_TAU = 1.0


_H = _N // 2


def _glcn_body(x_ref, xt_ref, w_ref, a_ref, p_ref):
    x = x_ref[0]
    xt = xt_ref[0]
    a00 = jnp.zeros((_H, _H), jnp.float32)
    a01 = jnp.zeros((_H, _H), jnp.float32)
    a11 = jnp.zeros((_H, _H), jnp.float32)
    for k in range(_K):
        c0 = x[0:_H, k:k + 1]
        c1 = x[_H:_N, k:k + 1]
        r0 = xt[k:k + 1, 0:_H]
        r1 = xt[k:k + 1, _H:_N]
        wk = w_ref[k, 0]
        d00 = jnp.abs(c0 - r0).astype(jnp.bfloat16).astype(jnp.float32)
        d01 = jnp.abs(c0 - r1).astype(jnp.bfloat16).astype(jnp.float32)
        d11 = jnp.abs(c1 - r1).astype(jnp.bfloat16).astype(jnp.float32)
        a00 = a00 + wk * d00
        a01 = a01 + wk * d01
        a11 = a11 + wk * d11
    ii = jax.lax.broadcasted_iota(jnp.int32, (_H, _H), 0)
    jj = jax.lax.broadcasted_iota(jnp.int32, (_H, _H), 1)
    diag = ii == jj

    def _finish(acc, on_diag):
        y = jax.nn.sigmoid(acc / _TAU)
        hard = y > 0.5
        plog = jnp.log(jnp.where(hard, y, 1.0 - y) + 1e-8)
        if on_diag:
            a_blk = jnp.where(diag | hard, 1.0, 0.0).astype(jnp.float32)
            s = jnp.sum(jnp.where(diag, 0.0, plog))
        else:
            a_blk = jnp.where(hard, 1.0, 0.0).astype(jnp.float32)
            s = jnp.sum(plog)
        return a_blk, s

    A00, s00 = _finish(a00, True)
    A01, s01 = _finish(a01, False)
    A11, s11 = _finish(a11, True)
    a_ref[0, 0:_H, 0:_H] = A00
    a_ref[0, 0:_H, _H:_N] = A01
    a_ref[0, _H:_N, 0:_H] = jnp.transpose(A01)
    a_ref[0, _H:_N, _H:_N] = A11
    p_ref[...] = (s00 + s11 + 2.0 * s01).reshape(1, 1, 1)


def kernel(h, a_link, rollout):
    hf = jax.lax.stop_gradient(h[:, :, :_K])
    hft = jnp.transpose(hf, (0, 2, 1))
    wi = jax.lax.bitcast_convert_type(a_link, jnp.int32)
    wlsb = jax.lax.shift_right_logical(wi, 16) & 1
    wi = (wi + 0x7FFF + wlsb) & jnp.int32(-65536)
    w_r = jax.lax.bitcast_convert_type(wi, jnp.float32)
    b = h.shape[0]
    a_out, probs = pl.pallas_call(
        _glcn_body,
        grid=(b,),
        in_specs=[
            pl.BlockSpec((1, _N, _K), lambda i: (i, 0, 0)),
            pl.BlockSpec((1, _K, _N), lambda i: (i, 0, 0)),
            pl.BlockSpec(memory_space=pltpu.SMEM),
        ],
        out_specs=[
            pl.BlockSpec((1, _N, _N), lambda i: (i, 0, 0)),
            pl.BlockSpec((1, 1, 1), lambda i: (i, 0, 0)),
        ],
        out_shape=[
            jax.ShapeDtypeStruct((b, _N, _N), jnp.float32),
            jax.ShapeDtypeStruct((b, 1, 1), jnp.float32),
        ],
    )(hf, hft, w_r)
    return (a_out, probs[:, 0, 0])
